# Initial kernel scaffold; baseline (speedup 1.0000x reference)
import jax
import jax.numpy as jnp
from jax.experimental import pallas as pl


def kernel(x, expander_edge_attr, expander_edge_index, batch_vec, W_Q, W_K, W_E, W_V, g1, beta1, W1, b1, W2, b2, g2, beta2):
    raise NotImplementedError("write your pallas kernel here")


# speedup vs baseline: 15.1105x; 15.1105x over previous
import functools

import jax
import jax.numpy as jnp
from jax import lax
from jax.experimental import pallas as pl
from jax.experimental.pallas import tpu as pltpu
from jax.experimental.pallas import tpu_sc as plsc

N = 10000
E = 320000
D = 128
H = 8
DH = 16
DE = 16
DFF = 2 * D

ACC_W = 144
N_ACC = 10240
NC = 2
NS = 16
NW = NC * NS
EPW = E // NW
CH = 40
NCHUNK = EPW // CH
ROWS_PER_TILE = N_ACC // NS
WB = 40
NWB = ROWS_PER_TILE // WB


def _proj_body(x_ref, wkv_ref, wq_ref, kv_ref, q_ref):
    xb = x_ref[...]
    kv_ref[...] = jnp.dot(xb, wkv_ref[...], preferred_element_type=jnp.float32)
    q_ref[...] = jnp.dot(xb, wq_ref[...], preferred_element_type=jnp.float32) * 0.25


def _proj(x, w_kv, w_q):
    bn = 1000
    return pl.pallas_call(
        _proj_body,
        grid=(N // bn,),
        in_specs=[
            pl.BlockSpec((bn, D), lambda i: (i, 0)),
            pl.BlockSpec((D, 2 * D), lambda i: (0, 0)),
            pl.BlockSpec((D, D), lambda i: (0, 0)),
        ],
        out_specs=[
            pl.BlockSpec((bn, 2 * D), lambda i: (i, 0)),
            pl.BlockSpec((bn, D), lambda i: (i, 0)),
        ],
        out_shape=[
            jax.ShapeDtypeStruct((N, 2 * D), jnp.float32),
            jax.ShapeDtypeStruct((N, D), jnp.float32),
        ],
    )(x, w_kv, w_q)


def _emat_body(a_ref, we_ref, e_ref):
    e_ref[...] = jnp.dot(a_ref[...], we_ref[...], preferred_element_type=jnp.float32)


def _emat(attr, w_e):
    be = 8000
    return pl.pallas_call(
        _emat_body,
        grid=(E // be,),
        in_specs=[
            pl.BlockSpec((be, DE), lambda i: (i, 0)),
            pl.BlockSpec((DE, D), lambda i: (0, 0)),
        ],
        out_specs=pl.BlockSpec((be, D), lambda i: (i, 0)),
        out_shape=jax.ShapeDtypeStruct((E, D), jnp.float32),
    )(attr, w_e)


def _sc_attention(kv, qs, emat, src, dst):
    mesh = plsc.VectorSubcoreMesh(core_axis_name="c", subcore_axis_name="s")

    @functools.partial(
        pl.kernel,
        mesh=mesh,
        out_type=jax.ShapeDtypeStruct((NC, N_ACC, ACC_W), jnp.float32),
        scratch_types=[
            pltpu.VMEM((CH,), jnp.int32),
            pltpu.VMEM((CH,), jnp.int32),
            pltpu.VMEM((CH, 2 * D), jnp.float32),
            pltpu.VMEM((CH, D), jnp.float32),
            pltpu.VMEM((CH, D), jnp.float32),
            pltpu.VMEM((CH, ACC_W), jnp.float32),
            pltpu.VMEM_SHARED((N_ACC, ACC_W), jnp.float32),
            pltpu.SemaphoreType.DMA,
            pltpu.SemaphoreType.DMA,
            pltpu.SemaphoreType.DMA,
        ],
        compiler_params=pltpu.CompilerParams(
            needs_layout_passes=False, use_tc_tiling_on_sc=False),
    )
    def attn(kv_hbm, q_hbm, emat_hbm, src_hbm, dst_hbm, out_hbm,
             src_v, dst_v, kv_v, q_v, e_v, msg_v, acc_sh,
             sem0, sem1, sem2):
        c = lax.axis_index("c")
        s = lax.axis_index("s")
        wid = c * NS + s
        row0 = s * ROWS_PER_TILE

        zeros16 = jnp.zeros((16,), jnp.float32)

        def zero_stage(i, carry):
            r = i // (ACC_W // 16)
            col = lax.rem(i, ACC_W // 16) * 16
            msg_v[r, pl.ds(col, 16)] = zeros16
            return carry

        lax.fori_loop(0, WB * (ACC_W // 16), zero_stage, 0)

        def zero_acc(j, carry):
            pltpu.sync_copy(msg_v, acc_sh.at[pl.ds(row0 + j * WB, WB)])
            return carry

        lax.fori_loop(0, NWB, zero_acc, 0)
        plsc.subcore_barrier()

        lanes = lax.iota(jnp.int32, 16)
        ebase = wid * EPW

        def chunk_body(i, carry):
            base = ebase + i * CH
            pltpu.sync_copy(src_hbm.at[pl.ds(base, CH)], src_v)
            pltpu.sync_copy(dst_hbm.at[pl.ds(base, CH)], dst_v)
            cp_kv = pltpu.async_copy(kv_hbm.at[src_v], kv_v, sem0)
            cp_q = pltpu.async_copy(q_hbm.at[dst_v], q_v, sem1)
            cp_e = pltpu.async_copy(emat_hbm.at[pl.ds(base, CH)], e_v, sem2)
            cp_kv.wait()
            cp_q.wait()
            cp_e.wait()

            def edge_body(e, ecarry):
                zrow = jnp.zeros((16,), jnp.float32)
                for h in range(H):
                    vk = kv_v[e, pl.ds(h * DH, DH)]
                    vq = q_v[e, pl.ds(h * DH, DH)]
                    ve = e_v[e, pl.ds(h * DH, DH)]
                    t = vk * vq * ve
                    sc = jnp.sum(t)
                    sv = jnp.broadcast_to(sc, (16,))
                    sv = jnp.exp(jnp.minimum(jnp.maximum(sv, -5.0), 5.0))
                    vv = kv_v[e, pl.ds(D + h * DH, DH)]
                    msg_v[e, pl.ds(h * DH, DH)] = vv * sv
                    zrow = jnp.where(lanes == h, sv, zrow)
                msg_v[e, pl.ds(D, 16)] = zrow
                return ecarry

            lax.fori_loop(0, CH, edge_body, 0)
            pltpu.sync_copy(msg_v, acc_sh.at[dst_v], add=True)
            return carry

        lax.fori_loop(0, NCHUNK, chunk_body, 0)
        plsc.subcore_barrier()

        def writeback(j, carry):
            pltpu.sync_copy(acc_sh.at[pl.ds(row0 + j * WB, WB)], msg_v)
            pltpu.sync_copy(msg_v, out_hbm.at[c, pl.ds(row0 + j * WB, WB)])
            return carry

        lax.fori_loop(0, NWB, writeback, 0)

    return attn(kv, qs, emat, src, dst)


def _final_body(p_ref, x_ref, r_ref, g1_ref, beta1_ref, w1_ref, b1_ref,
                w2_ref, b2_ref, g2_ref, beta2_ref, out_ref):
    acc = p_ref[0] + p_ref[1]
    w_v = acc[:, :D]
    z = acc[:, D:D + H]
    zfull = jnp.dot(z, r_ref[...], preferred_element_type=jnp.float32)
    h_att = w_v / (zfull + 1e-6)
    h = x_ref[...] + h_att
    mu = jnp.mean(h, axis=-1, keepdims=True)
    dlt = h - mu
    var = jnp.mean(dlt * dlt, axis=-1, keepdims=True)
    h = dlt / jnp.sqrt(var + 1e-5) * g1_ref[...] + beta1_ref[...]
    h_in2 = h
    h = jnp.maximum(jnp.dot(h, w1_ref[...], preferred_element_type=jnp.float32)
                    + b1_ref[...], 0.0)
    h = jnp.dot(h, w2_ref[...], preferred_element_type=jnp.float32) + b2_ref[...]
    h = h_in2 + h
    mu = jnp.mean(h, axis=-1, keepdims=True)
    dlt = h - mu
    var = jnp.mean(dlt * dlt, axis=-1, keepdims=True)
    out_ref[...] = dlt / jnp.sqrt(var + 1e-5) * g2_ref[...] + beta2_ref[...]


def _final(partial, x, rmat, g1, beta1, w1, b1, w2, b2, g2, beta2):
    bn = 1000
    return pl.pallas_call(
        _final_body,
        grid=(N // bn,),
        in_specs=[
            pl.BlockSpec((NC, bn, ACC_W), lambda i: (0, i, 0)),
            pl.BlockSpec((bn, D), lambda i: (i, 0)),
            pl.BlockSpec((H, D), lambda i: (0, 0)),
            pl.BlockSpec((1, D), lambda i: (0, 0)),
            pl.BlockSpec((1, D), lambda i: (0, 0)),
            pl.BlockSpec((D, DFF), lambda i: (0, 0)),
            pl.BlockSpec((1, DFF), lambda i: (0, 0)),
            pl.BlockSpec((DFF, D), lambda i: (0, 0)),
            pl.BlockSpec((1, D), lambda i: (0, 0)),
            pl.BlockSpec((1, D), lambda i: (0, 0)),
            pl.BlockSpec((1, D), lambda i: (0, 0)),
        ],
        out_specs=pl.BlockSpec((bn, D), lambda i: (i, 0)),
        out_shape=jax.ShapeDtypeStruct((N, D), jnp.float32),
    )(partial, x, rmat, g1, beta1, w1, b1, w2, b2, g2, beta2)


def kernel(x, expander_edge_attr, expander_edge_index, batch_vec,
           W_Q, W_K, W_E, W_V, g1, beta1, W1, b1, W2, b2, g2, beta2):
    del batch_vec
    src = expander_edge_index[0]
    dst = expander_edge_index[1]
    w_kv = jnp.concatenate([W_K, W_V], axis=1)

    kv, qs = _proj(x, w_kv, W_Q)
    emat = _emat(expander_edge_attr, W_E)
    partial = _sc_attention(kv, qs, emat, src, dst)[:, :N, :]

    rmat = jnp.repeat(jnp.eye(H, dtype=jnp.float32), DH, axis=1)
    return _final(partial, x, rmat,
                  g1.reshape(1, D), beta1.reshape(1, D),
                  W1, b1.reshape(1, DFF), W2, b2.reshape(1, D),
                  g2.reshape(1, D), beta2.reshape(1, D))


# speedup vs baseline: 35.8881x; 2.3750x over previous
import functools

import jax
import jax.numpy as jnp
from jax import lax
from jax.experimental import pallas as pl
from jax.experimental.pallas import tpu as pltpu
from jax.experimental.pallas import tpu_sc as plsc

N = 10000
E = 320000
D = 128
H = 8
DH = 16
DE = 16
DFF = 2 * D

ACC_W = 144
N_ACC = 10240
NC = 2
NS = 16
NW = NC * NS
EPW = E // NW
CH = 40
NCHUNK = EPW // CH
ROWS_PER_TILE = N_ACC // NS
WB = 40
NWB = ROWS_PER_TILE // WB


def _proj_body(x_ref, wkv_ref, wq_ref, kv_ref, q_ref):
    xb = x_ref[...]
    kv_ref[...] = jnp.dot(xb, wkv_ref[...], preferred_element_type=jnp.float32)
    q_ref[...] = jnp.dot(xb, wq_ref[...], preferred_element_type=jnp.float32) * 0.25


def _proj(x, w_kv, w_q):
    bn = 1000
    return pl.pallas_call(
        _proj_body,
        grid=(N // bn,),
        in_specs=[
            pl.BlockSpec((bn, D), lambda i: (i, 0)),
            pl.BlockSpec((D, 2 * D), lambda i: (0, 0)),
            pl.BlockSpec((D, D), lambda i: (0, 0)),
        ],
        out_specs=[
            pl.BlockSpec((bn, 2 * D), lambda i: (i, 0)),
            pl.BlockSpec((bn, D), lambda i: (i, 0)),
        ],
        out_shape=[
            jax.ShapeDtypeStruct((N, 2 * D), jnp.float32),
            jax.ShapeDtypeStruct((N, D), jnp.float32),
        ],
    )(x, w_kv, w_q)


def _emat_body(a_ref, we_ref, e_ref):
    e_ref[...] = jnp.dot(a_ref[...], we_ref[...], preferred_element_type=jnp.float32)


def _emat(attr, w_e):
    be = 8000
    return pl.pallas_call(
        _emat_body,
        grid=(E // be,),
        in_specs=[
            pl.BlockSpec((be, DE), lambda i: (i, 0)),
            pl.BlockSpec((DE, D), lambda i: (0, 0)),
        ],
        out_specs=pl.BlockSpec((be, D), lambda i: (i, 0)),
        out_shape=jax.ShapeDtypeStruct((E, D), jnp.float32),
    )(attr, w_e)


def _sc_attention(kv, qs, emat, src, dst):
    mesh = plsc.VectorSubcoreMesh(core_axis_name="c", subcore_axis_name="s")

    @functools.partial(
        pl.kernel,
        mesh=mesh,
        out_type=jax.ShapeDtypeStruct((NC, N_ACC, ACC_W), jnp.float32),
        scratch_types=[
            pltpu.VMEM((CH,), jnp.int32),
            pltpu.VMEM((CH,), jnp.int32),
            pltpu.VMEM((CH, 2 * D), jnp.float32),
            pltpu.VMEM((CH, D), jnp.float32),
            pltpu.VMEM((CH, D), jnp.float32),
            pltpu.VMEM((CH, ACC_W), jnp.float32),
            pltpu.VMEM_SHARED((N_ACC, ACC_W), jnp.float32),
            pltpu.SemaphoreType.DMA,
            pltpu.SemaphoreType.DMA,
            pltpu.SemaphoreType.DMA,
        ],
        compiler_params=pltpu.CompilerParams(
            needs_layout_passes=False, use_tc_tiling_on_sc=False),
    )
    def attn(kv_hbm, q_hbm, emat_hbm, src_hbm, dst_hbm, out_hbm,
             src_v, dst_v, kv_v, q_v, e_v, msg_v, acc_sh,
             sem0, sem1, sem2):
        c = lax.axis_index("c")
        s = lax.axis_index("s")
        wid = c * NS + s
        row0 = s * ROWS_PER_TILE

        zeros16 = jnp.zeros((16,), jnp.float32)

        def zero_stage(i, carry):
            r = i // (ACC_W // 16)
            col = lax.rem(i, ACC_W // 16) * 16
            msg_v[r, pl.ds(col, 16)] = zeros16
            return carry

        lax.fori_loop(0, WB * (ACC_W // 16), zero_stage, 0)

        def zero_acc(j, carry):
            pltpu.sync_copy(msg_v, acc_sh.at[pl.ds(row0 + j * WB, WB)])
            return carry

        lax.fori_loop(0, NWB, zero_acc, 0)
        plsc.subcore_barrier()

        lanes = lax.iota(jnp.int32, 16)
        ebase = wid * EPW

        def chunk_body(i, carry):
            base = ebase + i * CH
            pltpu.sync_copy(src_hbm.at[pl.ds(base, CH)], src_v)
            pltpu.sync_copy(dst_hbm.at[pl.ds(base, CH)], dst_v)
            cp_kv = pltpu.async_copy(kv_hbm.at[src_v], kv_v, sem0)
            cp_q = pltpu.async_copy(q_hbm.at[dst_v], q_v, sem1)
            cp_e = pltpu.async_copy(emat_hbm.at[pl.ds(base, CH)], e_v, sem2)
            cp_kv.wait()
            cp_q.wait()
            cp_e.wait()

            def edge_body(e):
                zrow = jnp.zeros((16,), jnp.float32)
                for h in range(H):
                    vk = kv_v[e, pl.ds(h * DH, DH)]
                    vq = q_v[e, pl.ds(h * DH, DH)]
                    ve = e_v[e, pl.ds(h * DH, DH)]
                    t = vk * vq * ve
                    sc = jnp.sum(t)
                    sv = jnp.broadcast_to(sc, (16,))
                    sv = jnp.exp(jnp.minimum(jnp.maximum(sv, -5.0), 5.0))
                    vv = kv_v[e, pl.ds(D + h * DH, DH)]
                    msg_v[e, pl.ds(h * DH, DH)] = vv * sv
                    zrow = jnp.where(lanes == h, sv, zrow)
                msg_v[e, pl.ds(D, 16)] = zrow

            plsc.parallel_loop(0, CH, 1, unroll=4)(edge_body)
            pltpu.sync_copy(msg_v, acc_sh.at[dst_v], add=True)
            return carry

        lax.fori_loop(0, NCHUNK, chunk_body, 0)
        plsc.subcore_barrier()

        def writeback(j, carry):
            pltpu.sync_copy(acc_sh.at[pl.ds(row0 + j * WB, WB)], msg_v)
            pltpu.sync_copy(msg_v, out_hbm.at[c, pl.ds(row0 + j * WB, WB)])
            return carry

        lax.fori_loop(0, NWB, writeback, 0)

    return attn(kv, qs, emat, src, dst)


def _final_body(p_ref, x_ref, r_ref, g1_ref, beta1_ref, w1_ref, b1_ref,
                w2_ref, b2_ref, g2_ref, beta2_ref, out_ref):
    acc = p_ref[0] + p_ref[1]
    w_v = acc[:, :D]
    z = acc[:, D:D + H]
    zfull = jnp.dot(z, r_ref[...], preferred_element_type=jnp.float32)
    h_att = w_v / (zfull + 1e-6)
    h = x_ref[...] + h_att
    mu = jnp.mean(h, axis=-1, keepdims=True)
    dlt = h - mu
    var = jnp.mean(dlt * dlt, axis=-1, keepdims=True)
    h = dlt / jnp.sqrt(var + 1e-5) * g1_ref[...] + beta1_ref[...]
    h_in2 = h
    h = jnp.maximum(jnp.dot(h, w1_ref[...], preferred_element_type=jnp.float32)
                    + b1_ref[...], 0.0)
    h = jnp.dot(h, w2_ref[...], preferred_element_type=jnp.float32) + b2_ref[...]
    h = h_in2 + h
    mu = jnp.mean(h, axis=-1, keepdims=True)
    dlt = h - mu
    var = jnp.mean(dlt * dlt, axis=-1, keepdims=True)
    out_ref[...] = dlt / jnp.sqrt(var + 1e-5) * g2_ref[...] + beta2_ref[...]


def _final(partial, x, rmat, g1, beta1, w1, b1, w2, b2, g2, beta2):
    bn = 1000
    return pl.pallas_call(
        _final_body,
        grid=(N // bn,),
        in_specs=[
            pl.BlockSpec((NC, bn, ACC_W), lambda i: (0, i, 0)),
            pl.BlockSpec((bn, D), lambda i: (i, 0)),
            pl.BlockSpec((H, D), lambda i: (0, 0)),
            pl.BlockSpec((1, D), lambda i: (0, 0)),
            pl.BlockSpec((1, D), lambda i: (0, 0)),
            pl.BlockSpec((D, DFF), lambda i: (0, 0)),
            pl.BlockSpec((1, DFF), lambda i: (0, 0)),
            pl.BlockSpec((DFF, D), lambda i: (0, 0)),
            pl.BlockSpec((1, D), lambda i: (0, 0)),
            pl.BlockSpec((1, D), lambda i: (0, 0)),
            pl.BlockSpec((1, D), lambda i: (0, 0)),
        ],
        out_specs=pl.BlockSpec((bn, D), lambda i: (i, 0)),
        out_shape=jax.ShapeDtypeStruct((N, D), jnp.float32),
    )(partial, x, rmat, g1, beta1, w1, b1, w2, b2, g2, beta2)


def kernel(x, expander_edge_attr, expander_edge_index, batch_vec,
           W_Q, W_K, W_E, W_V, g1, beta1, W1, b1, W2, b2, g2, beta2):
    del batch_vec
    src = expander_edge_index[0]
    dst = expander_edge_index[1]
    w_kv = jnp.concatenate([W_K, W_V], axis=1)

    kv, qs = _proj(x, w_kv, W_Q)
    emat = _emat(expander_edge_attr, W_E)
    partial = _sc_attention(kv, qs, emat, src, dst)[:, :N, :]

    rmat = jnp.repeat(jnp.eye(H, dtype=jnp.float32), DH, axis=1)
    return _final(partial, x, rmat,
                  g1.reshape(1, D), beta1.reshape(1, D),
                  W1, b1.reshape(1, DFF), W2, b2.reshape(1, D),
                  g2.reshape(1, D), beta2.reshape(1, D))


# speedup vs baseline: 36.9598x; 1.0299x over previous
import functools

import jax
import jax.numpy as jnp
from jax import lax
from jax.experimental import pallas as pl
from jax.experimental.pallas import tpu as pltpu
from jax.experimental.pallas import tpu_sc as plsc

N = 10000
E = 320000
D = 128
H = 8
DH = 16
DE = 16
DFF = 2 * D

ACC_W = 144
N_ACC = 10240
NC = 2
NS = 16
NW = NC * NS
EPW = E // NW
CH = 40
NCHUNK = EPW // CH
ROWS_PER_TILE = N_ACC // NS
WB = 40
NWB = ROWS_PER_TILE // WB


def _proj_body(x_ref, wkv_ref, wq_ref, kv_ref, q_ref):
    xb = x_ref[...]
    kv_ref[...] = jnp.dot(xb, wkv_ref[...], preferred_element_type=jnp.float32)
    q_ref[...] = jnp.dot(xb, wq_ref[...], preferred_element_type=jnp.float32) * 0.25


def _proj(x, w_kv, w_q):
    bn = 1000
    return pl.pallas_call(
        _proj_body,
        grid=(N // bn,),
        in_specs=[
            pl.BlockSpec((bn, D), lambda i: (i, 0)),
            pl.BlockSpec((D, 2 * D), lambda i: (0, 0)),
            pl.BlockSpec((D, D), lambda i: (0, 0)),
        ],
        out_specs=[
            pl.BlockSpec((bn, 2 * D), lambda i: (i, 0)),
            pl.BlockSpec((bn, D), lambda i: (i, 0)),
        ],
        out_shape=[
            jax.ShapeDtypeStruct((N, 2 * D), jnp.float32),
            jax.ShapeDtypeStruct((N, D), jnp.float32),
        ],
    )(x, w_kv, w_q)


def _emat_body(a_ref, we_ref, e_ref):
    e_ref[...] = jnp.dot(a_ref[...], we_ref[...], preferred_element_type=jnp.float32)


def _emat(attr, w_e):
    be = 8000
    return pl.pallas_call(
        _emat_body,
        grid=(E // be,),
        in_specs=[
            pl.BlockSpec((be, DE), lambda i: (i, 0)),
            pl.BlockSpec((DE, D), lambda i: (0, 0)),
        ],
        out_specs=pl.BlockSpec((be, D), lambda i: (i, 0)),
        out_shape=jax.ShapeDtypeStruct((E, D), jnp.float32),
    )(attr, w_e)


def _sc_attention(kv, qs, emat, src, dst):
    mesh = plsc.VectorSubcoreMesh(core_axis_name="c", subcore_axis_name="s")

    @functools.partial(
        pl.kernel,
        mesh=mesh,
        out_type=jax.ShapeDtypeStruct((NC, N_ACC, ACC_W), jnp.float32),
        scratch_types=[
            pltpu.VMEM((2, CH), jnp.int32),
            pltpu.VMEM((2, CH), jnp.int32),
            pltpu.VMEM((2, CH, 2 * D), jnp.float32),
            pltpu.VMEM((CH, D), jnp.float32),
            pltpu.VMEM((CH, D), jnp.float32),
            pltpu.VMEM((CH, ACC_W), jnp.float32),
            pltpu.VMEM_SHARED((N_ACC, ACC_W), jnp.float32),
            pltpu.SemaphoreType.DMA,
            pltpu.SemaphoreType.DMA,
            pltpu.SemaphoreType.DMA,
            pltpu.SemaphoreType.DMA,
        ],
        compiler_params=pltpu.CompilerParams(
            needs_layout_passes=False, use_tc_tiling_on_sc=False),
    )
    def attn(kv_hbm, q_hbm, emat_hbm, src_hbm, dst_hbm, out_hbm,
             src_v, dst_v, kv_v, q_v, e_v, msg_v, acc_sh,
             sem_kv0, sem_kv1, sem_q, sem_e):
        c = lax.axis_index("c")
        s = lax.axis_index("s")
        wid = c * NS + s
        row0 = s * ROWS_PER_TILE

        zeros16 = jnp.zeros((16,), jnp.float32)

        def zero_stage(i, carry):
            r = i // (ACC_W // 16)
            col = lax.rem(i, ACC_W // 16) * 16
            msg_v[r, pl.ds(col, 16)] = zeros16
            return carry

        lax.fori_loop(0, WB * (ACC_W // 16), zero_stage, 0)

        def zero_acc(j, carry):
            pltpu.sync_copy(msg_v, acc_sh.at[pl.ds(row0 + j * WB, WB)])
            return carry

        lax.fori_loop(0, NWB, zero_acc, 0)
        plsc.subcore_barrier()

        lanes = lax.iota(jnp.int32, 16)
        ebase = wid * EPW
        kv_sems = (sem_kv0, sem_kv1)

        pltpu.sync_copy(src_hbm.at[pl.ds(ebase, CH)], src_v.at[0])
        pltpu.sync_copy(dst_hbm.at[pl.ds(ebase, CH)], dst_v.at[0])
        pltpu.async_copy(kv_hbm.at[src_v.at[0]], kv_v.at[0], sem_kv0)

        def pair_body(i2, carry):
            for b in range(2):
                cid = 2 * i2 + b
                base = ebase + cid * CH
                cp_q = pltpu.async_copy(q_hbm.at[dst_v.at[b]], q_v, sem_q)
                cp_e = pltpu.async_copy(emat_hbm.at[pl.ds(base, CH)], e_v,
                                        sem_e)
                nxt = base + CH

                @pl.when(cid + 1 < NCHUNK)
                def _():
                    pltpu.sync_copy(src_hbm.at[pl.ds(nxt, CH)],
                                    src_v.at[1 - b])
                    pltpu.sync_copy(dst_hbm.at[pl.ds(nxt, CH)],
                                    dst_v.at[1 - b])
                    pltpu.async_copy(kv_hbm.at[src_v.at[1 - b]],
                                     kv_v.at[1 - b], kv_sems[1 - b])

                pltpu.make_async_copy(kv_hbm.at[src_v.at[b]], kv_v.at[b],
                                      kv_sems[b]).wait()
                cp_q.wait()
                cp_e.wait()
                _compute_chunk(b)
            return carry

        def _compute_chunk(b):

            def edge_body(e):
                zrow = jnp.zeros((16,), jnp.float32)
                for h in range(H):
                    vk = kv_v[b, e, pl.ds(h * DH, DH)]
                    vq = q_v[e, pl.ds(h * DH, DH)]
                    ve = e_v[e, pl.ds(h * DH, DH)]
                    t = vk * vq * ve
                    sc = jnp.sum(t)
                    sc = jnp.minimum(jnp.maximum(sc, -5.0), 5.0)
                    sv = jnp.exp(jnp.broadcast_to(sc, (16,)))
                    vv = kv_v[b, e, pl.ds(D + h * DH, DH)]
                    msg_v[e, pl.ds(h * DH, DH)] = vv * sv
                    zrow = jnp.where(lanes == h, sv, zrow)
                msg_v[e, pl.ds(D, 16)] = zrow

            plsc.parallel_loop(0, CH, 1, unroll=4)(edge_body)
            pltpu.sync_copy(msg_v, acc_sh.at[dst_v.at[b]], add=True)

        lax.fori_loop(0, NCHUNK // 2, pair_body, 0)
        plsc.subcore_barrier()

        def writeback(j, carry):
            pltpu.sync_copy(acc_sh.at[pl.ds(row0 + j * WB, WB)], msg_v)
            pltpu.sync_copy(msg_v, out_hbm.at[c, pl.ds(row0 + j * WB, WB)])
            return carry

        lax.fori_loop(0, NWB, writeback, 0)

    return attn(kv, qs, emat, src, dst)


def _final_body(p_ref, x_ref, r_ref, g1_ref, beta1_ref, w1_ref, b1_ref,
                w2_ref, b2_ref, g2_ref, beta2_ref, out_ref):
    acc = p_ref[0] + p_ref[1]
    w_v = acc[:, :D]
    z = acc[:, D:D + H]
    zfull = jnp.dot(z, r_ref[...], preferred_element_type=jnp.float32)
    h_att = w_v / (zfull + 1e-6)
    h = x_ref[...] + h_att
    mu = jnp.mean(h, axis=-1, keepdims=True)
    dlt = h - mu
    var = jnp.mean(dlt * dlt, axis=-1, keepdims=True)
    h = dlt / jnp.sqrt(var + 1e-5) * g1_ref[...] + beta1_ref[...]
    h_in2 = h
    h = jnp.maximum(jnp.dot(h, w1_ref[...], preferred_element_type=jnp.float32)
                    + b1_ref[...], 0.0)
    h = jnp.dot(h, w2_ref[...], preferred_element_type=jnp.float32) + b2_ref[...]
    h = h_in2 + h
    mu = jnp.mean(h, axis=-1, keepdims=True)
    dlt = h - mu
    var = jnp.mean(dlt * dlt, axis=-1, keepdims=True)
    out_ref[...] = dlt / jnp.sqrt(var + 1e-5) * g2_ref[...] + beta2_ref[...]


def _final(partial, x, rmat, g1, beta1, w1, b1, w2, b2, g2, beta2):
    bn = 1000
    return pl.pallas_call(
        _final_body,
        grid=(N // bn,),
        in_specs=[
            pl.BlockSpec((NC, bn, ACC_W), lambda i: (0, i, 0)),
            pl.BlockSpec((bn, D), lambda i: (i, 0)),
            pl.BlockSpec((H, D), lambda i: (0, 0)),
            pl.BlockSpec((1, D), lambda i: (0, 0)),
            pl.BlockSpec((1, D), lambda i: (0, 0)),
            pl.BlockSpec((D, DFF), lambda i: (0, 0)),
            pl.BlockSpec((1, DFF), lambda i: (0, 0)),
            pl.BlockSpec((DFF, D), lambda i: (0, 0)),
            pl.BlockSpec((1, D), lambda i: (0, 0)),
            pl.BlockSpec((1, D), lambda i: (0, 0)),
            pl.BlockSpec((1, D), lambda i: (0, 0)),
        ],
        out_specs=pl.BlockSpec((bn, D), lambda i: (i, 0)),
        out_shape=jax.ShapeDtypeStruct((N, D), jnp.float32),
    )(partial, x, rmat, g1, beta1, w1, b1, w2, b2, g2, beta2)


def kernel(x, expander_edge_attr, expander_edge_index, batch_vec,
           W_Q, W_K, W_E, W_V, g1, beta1, W1, b1, W2, b2, g2, beta2):
    del batch_vec
    src = expander_edge_index[0]
    dst = expander_edge_index[1]
    w_kv = jnp.concatenate([W_K, W_V], axis=1)

    kv, qs = _proj(x, w_kv, W_Q)
    emat = _emat(expander_edge_attr, W_E)
    partial = _sc_attention(kv, qs, emat, src, dst)[:, :N, :]

    rmat = jnp.repeat(jnp.eye(H, dtype=jnp.float32), DH, axis=1)
    return _final(partial, x, rmat,
                  g1.reshape(1, D), beta1.reshape(1, D),
                  W1, b1.reshape(1, DFF), W2, b2.reshape(1, D),
                  g2.reshape(1, D), beta2.reshape(1, D))


# speedup vs baseline: 38.5459x; 1.0429x over previous
import functools

import jax
import jax.numpy as jnp
from jax import lax
from jax.experimental import pallas as pl
from jax.experimental.pallas import tpu as pltpu
from jax.experimental.pallas import tpu_sc as plsc

N = 10000
E = 320000
D = 128
H = 8
DH = 16
DE = 16
DFF = 2 * D

ACC_W = 144
N_ACC = N
NC = 2
NS = 16
NW = NC * NS
EPW = E // NW
CH = 40
NCHUNK = EPW // CH
WB = 40
NPIECE = N_ACC // WB


def _proj_body(x_ref, wkv_ref, wq_ref, kv_ref, q_ref):
    xb = x_ref[...]
    kv = jnp.dot(xb, wkv_ref[...], preferred_element_type=jnp.float32)
    kv_ref[...] = kv.astype(jnp.bfloat16)
    q = jnp.dot(xb, wq_ref[...], preferred_element_type=jnp.float32) * 0.25
    q_ref[...] = q.astype(jnp.bfloat16)


def _proj(x, w_kv, w_q):
    bn = 2000
    return pl.pallas_call(
        _proj_body,
        grid=(N // bn,),
        in_specs=[
            pl.BlockSpec((bn, D), lambda i: (i, 0)),
            pl.BlockSpec((D, 2 * D), lambda i: (0, 0)),
            pl.BlockSpec((D, D), lambda i: (0, 0)),
        ],
        out_specs=[
            pl.BlockSpec((bn, 2 * D), lambda i: (i, 0)),
            pl.BlockSpec((bn, D), lambda i: (i, 0)),
        ],
        out_shape=[
            jax.ShapeDtypeStruct((N, 2 * D), jnp.bfloat16),
            jax.ShapeDtypeStruct((N, D), jnp.bfloat16),
        ],
    )(x, w_kv, w_q)


def _emat_body(a_ref, we_ref, e_ref):
    e = jnp.dot(a_ref[...], we_ref[...], preferred_element_type=jnp.float32)
    e_ref[...] = e.astype(jnp.bfloat16)


def _emat(attr, w_e):
    be = 8000
    return pl.pallas_call(
        _emat_body,
        grid=(E // be,),
        in_specs=[
            pl.BlockSpec((be, DE), lambda i: (i, 0)),
            pl.BlockSpec((DE, D), lambda i: (0, 0)),
        ],
        out_specs=pl.BlockSpec((be, D), lambda i: (i, 0)),
        out_shape=jax.ShapeDtypeStruct((E, D), jnp.bfloat16),
    )(attr, w_e)


def _sc_attention(kv, qs, emat, src, dst):
    mesh = plsc.VectorSubcoreMesh(core_axis_name="c", subcore_axis_name="s")

    @functools.partial(
        pl.kernel,
        mesh=mesh,
        out_type=jax.ShapeDtypeStruct((NC, N_ACC, ACC_W), jnp.float32),
        scratch_types=[
            pltpu.VMEM((2, CH), jnp.int32),
            pltpu.VMEM((2, CH), jnp.int32),
            pltpu.VMEM((2, CH, 2 * D), jnp.bfloat16),
            pltpu.VMEM((2, CH, D), jnp.bfloat16),
            pltpu.VMEM((2, CH, D), jnp.bfloat16),
            pltpu.VMEM((2, CH, ACC_W), jnp.float32),
            pltpu.VMEM_SHARED((N_ACC, ACC_W), jnp.float32),
            pltpu.SemaphoreType.DMA,
            pltpu.SemaphoreType.DMA,
            pltpu.SemaphoreType.DMA,
            pltpu.SemaphoreType.DMA,
            pltpu.SemaphoreType.DMA,
            pltpu.SemaphoreType.DMA,
            pltpu.SemaphoreType.DMA,
            pltpu.SemaphoreType.DMA,
        ],
        compiler_params=pltpu.CompilerParams(
            needs_layout_passes=False, use_tc_tiling_on_sc=False),
    )
    def attn(kv_hbm, q_hbm, emat_hbm, src_hbm, dst_hbm, out_hbm,
             src_v, dst_v, kv_v, q_v, e_v, msg_v, acc_sh,
             sem_kv0, sem_kv1, sem_q0, sem_q1, sem_e0, sem_e1,
             sem_sc0, sem_sc1):
        c = lax.axis_index("c")
        s = lax.axis_index("s")
        wid = c * NS + s
        npieces = (NPIECE - 1 - s) // NS + 1

        zeros16 = jnp.zeros((16,), jnp.float32)

        def zero_stage(i, carry):
            r = i // (ACC_W // 16)
            col = lax.rem(i, ACC_W // 16) * 16
            msg_v[0, r, pl.ds(col, 16)] = zeros16
            return carry

        lax.fori_loop(0, WB * (ACC_W // 16), zero_stage, 0)

        def zero_acc(j, carry):
            row = (s + j * NS) * WB
            pltpu.sync_copy(msg_v.at[0], acc_sh.at[pl.ds(row, WB)])
            return carry

        lax.fori_loop(0, npieces, zero_acc, 0)
        plsc.subcore_barrier()

        lanes = lax.iota(jnp.int32, 16)
        ebase = wid * EPW
        kv_sems = (sem_kv0, sem_kv1)
        q_sems = (sem_q0, sem_q1)
        e_sems = (sem_e0, sem_e1)
        sc_sems = (sem_sc0, sem_sc1)

        pltpu.sync_copy(src_hbm.at[pl.ds(ebase, CH)], src_v.at[0])
        pltpu.sync_copy(dst_hbm.at[pl.ds(ebase, CH)], dst_v.at[0])
        pltpu.async_copy(kv_hbm.at[src_v.at[0]], kv_v.at[0], sem_kv0)
        pltpu.async_copy(q_hbm.at[dst_v.at[0]], q_v.at[0], sem_q0)
        pltpu.async_copy(emat_hbm.at[pl.ds(ebase, CH)], e_v.at[0], sem_e0)

        def pair_body(i2, carry):
            for b in range(2):
                cid = 2 * i2 + b
                base = ebase + cid * CH
                nxt = base + CH

                @pl.when(cid + 1 < NCHUNK)
                def _():
                    pltpu.sync_copy(src_hbm.at[pl.ds(nxt, CH)],
                                    src_v.at[1 - b])
                    pltpu.sync_copy(dst_hbm.at[pl.ds(nxt, CH)],
                                    dst_v.at[1 - b])
                    pltpu.async_copy(kv_hbm.at[src_v.at[1 - b]],
                                     kv_v.at[1 - b], kv_sems[1 - b])
                    pltpu.async_copy(q_hbm.at[dst_v.at[1 - b]],
                                     q_v.at[1 - b], q_sems[1 - b])
                    pltpu.async_copy(emat_hbm.at[pl.ds(nxt, CH)],
                                     e_v.at[1 - b], e_sems[1 - b])

                pltpu.make_async_copy(kv_hbm.at[src_v.at[b]], kv_v.at[b],
                                      kv_sems[b]).wait()
                pltpu.make_async_copy(q_hbm.at[dst_v.at[b]], q_v.at[b],
                                      q_sems[b]).wait()
                pltpu.make_async_copy(emat_hbm.at[pl.ds(base, CH)],
                                      e_v.at[b], e_sems[b]).wait()

                @pl.when(cid >= 2)
                def _():
                    pltpu.make_async_copy(msg_v.at[b], acc_sh.at[dst_v.at[b]],
                                          sc_sems[b]).wait()

                _compute_chunk(b)
            return carry

        def _compute_chunk(b):

            def edge_body(e):
                zrow = jnp.zeros((16,), jnp.float32)
                for hp in range(H // 2):
                    ks = plsc.unpack(kv_v[b, e, pl.ds(hp * 2 * DH, 2 * DH)],
                                     format=plsc.PackFormat.INTERLEAVED)
                    qs2 = plsc.unpack(q_v[b, e, pl.ds(hp * 2 * DH, 2 * DH)],
                                      format=plsc.PackFormat.INTERLEAVED)
                    es = plsc.unpack(e_v[b, e, pl.ds(hp * 2 * DH, 2 * DH)],
                                     format=plsc.PackFormat.INTERLEAVED)
                    vs = plsc.unpack(
                        kv_v[b, e, pl.ds(D + hp * 2 * DH, 2 * DH)],
                        format=plsc.PackFormat.INTERLEAVED)
                    for j in range(2):
                        h = hp * 2 + j
                        t = ks[j] * qs2[j] * es[j]
                        sc = jnp.sum(t)
                        sc = jnp.minimum(jnp.maximum(sc, -5.0), 5.0)
                        sv = jnp.exp(jnp.broadcast_to(sc, (16,)))
                        msg_v[b, e, pl.ds(h * DH, DH)] = vs[j] * sv
                        zrow = jnp.where(lanes == h, sv, zrow)
                msg_v[b, e, pl.ds(D, 16)] = zrow

            plsc.parallel_loop(0, CH, 1, unroll=4)(edge_body)
            pltpu.async_copy(msg_v.at[b], acc_sh.at[dst_v.at[b]],
                             sc_sems[b], add=True)

        lax.fori_loop(0, NCHUNK // 2, pair_body, 0)

        for b in range(2):
            pltpu.make_async_copy(msg_v.at[b], acc_sh.at[dst_v.at[b]],
                                  sc_sems[b]).wait()
        plsc.subcore_barrier()

        def writeback(j, carry):
            row = (s + j * NS) * WB
            pltpu.sync_copy(acc_sh.at[pl.ds(row, WB)], msg_v.at[0])
            pltpu.sync_copy(msg_v.at[0], out_hbm.at[c, pl.ds(row, WB)])
            return carry

        lax.fori_loop(0, npieces, writeback, 0)

    return attn(kv, qs, emat, src, dst)


def _final_body(p_ref, x_ref, r_ref, g1_ref, beta1_ref, w1_ref, b1_ref,
                w2_ref, b2_ref, g2_ref, beta2_ref, out_ref):
    acc = p_ref[0] + p_ref[1]
    w_v = acc[:, :D]
    z = acc[:, D:D + H]
    zfull = jnp.dot(z, r_ref[...], preferred_element_type=jnp.float32)
    h_att = w_v / (zfull + 1e-6)
    h = x_ref[...] + h_att
    mu = jnp.mean(h, axis=-1, keepdims=True)
    dlt = h - mu
    var = jnp.mean(dlt * dlt, axis=-1, keepdims=True)
    h = dlt / jnp.sqrt(var + 1e-5) * g1_ref[...] + beta1_ref[...]
    h_in2 = h
    h = jnp.maximum(jnp.dot(h, w1_ref[...], preferred_element_type=jnp.float32)
                    + b1_ref[...], 0.0)
    h = jnp.dot(h, w2_ref[...], preferred_element_type=jnp.float32) + b2_ref[...]
    h = h_in2 + h
    mu = jnp.mean(h, axis=-1, keepdims=True)
    dlt = h - mu
    var = jnp.mean(dlt * dlt, axis=-1, keepdims=True)
    out_ref[...] = dlt / jnp.sqrt(var + 1e-5) * g2_ref[...] + beta2_ref[...]


def _final(partial, x, rmat, g1, beta1, w1, b1, w2, b2, g2, beta2):
    bn = 1000
    return pl.pallas_call(
        _final_body,
        grid=(N // bn,),
        in_specs=[
            pl.BlockSpec((NC, bn, ACC_W), lambda i: (0, i, 0)),
            pl.BlockSpec((bn, D), lambda i: (i, 0)),
            pl.BlockSpec((H, D), lambda i: (0, 0)),
            pl.BlockSpec((1, D), lambda i: (0, 0)),
            pl.BlockSpec((1, D), lambda i: (0, 0)),
            pl.BlockSpec((D, DFF), lambda i: (0, 0)),
            pl.BlockSpec((1, DFF), lambda i: (0, 0)),
            pl.BlockSpec((DFF, D), lambda i: (0, 0)),
            pl.BlockSpec((1, D), lambda i: (0, 0)),
            pl.BlockSpec((1, D), lambda i: (0, 0)),
            pl.BlockSpec((1, D), lambda i: (0, 0)),
        ],
        out_specs=pl.BlockSpec((bn, D), lambda i: (i, 0)),
        out_shape=jax.ShapeDtypeStruct((N, D), jnp.float32),
    )(partial, x, rmat, g1, beta1, w1, b1, w2, b2, g2, beta2)


def kernel(x, expander_edge_attr, expander_edge_index, batch_vec,
           W_Q, W_K, W_E, W_V, g1, beta1, W1, b1, W2, b2, g2, beta2):
    del batch_vec
    src = expander_edge_index[0]
    dst = expander_edge_index[1]
    w_kv = jnp.concatenate([W_K, W_V], axis=1)

    g = jnp.arange(0, D, 32)[:, None]
    pairs = jnp.stack([jnp.arange(16), jnp.arange(16) + 16],
                      axis=1).reshape(-1)
    perm128 = (g + pairs[None, :]).reshape(-1)
    perm256 = jnp.concatenate([perm128, perm128 + D])

    kv, qs = _proj(x, w_kv[:, perm256], W_Q[:, perm128])
    emat = _emat(expander_edge_attr, W_E[:, perm128])
    partial = _sc_attention(kv, qs, emat, src, dst)

    rmat = jnp.repeat(jnp.eye(H, dtype=jnp.float32), DH, axis=1)
    return _final(partial, x, rmat,
                  g1.reshape(1, D), beta1.reshape(1, D),
                  W1, b1.reshape(1, DFF), W2, b2.reshape(1, D),
                  g2.reshape(1, D), beta2.reshape(1, D))
